# baseline (speedup 1.0000x reference)
import functools

import jax
import jax.numpy as jnp
from jax import lax
from jax.experimental import pallas as pl
from jax.experimental.pallas import tpu as pltpu
from jax.experimental.pallas import tpu_sc as plsc

N_NODES = 10000
N_EDGES = 320000
D = 128
N_REL = 32
N_BASES = 8

NC = 2
NS = 16
L = 16

CHUNK = 1728
N_PASSES = 3
N_CHUNKS = 6
ROW_W = 32
ROWS_PER_CHUNK = CHUNK * N_REL
DUMMY_ROW = ROWS_PER_CHUNK
S_ROWS = N_CHUNKS * ROWS_PER_CHUNK

E_PAD = 327680
E_SLICE = E_PAD // NS
SB = 512
EB = 128
N_BLOCKS = E_SLICE // EB
BLK_PER_SB = SB // EB
N_SB = E_SLICE // SB
ROWS_PER_TILE = ROWS_PER_CHUNK // NS
ZROWS = 64

DST_SENTINEL = 1 << 20


def _u_tc_kernel(x_ref, wc_ref, u_ref):
    for q in range(4):
        u_ref[:, q * N_BASES:(q + 1) * N_BASES] = jnp.dot(
            x_ref[:, q * 32:(q + 1) * 32], wc_ref[...],
            preferred_element_type=jnp.float32)


def _h_tc_kernel(s_ref, w_ref, h_ref):
    h_ref[...] = jnp.dot(s_ref[...], w_ref[...],
                         preferred_element_type=jnp.float32)


def _sc_body(u_hbm, src_hbm, dst_hbm, rel_hbm, norm_hbm, zero_hbm, s_out,
             fsrc, fdst, frel, fnorm, rows, pay, sidx, gidx, zbuf,
             csrc, csidx, cnorm, cnt_ref, s_sh,
             fsem, ssem, gsem):
    c = lax.axis_index("c")
    s = lax.axis_index("s")
    ebase = pl.multiple_of(s * E_SLICE, SB)

    dummy16 = jnp.full((L,), DUMMY_ROW, jnp.int32)

    pltpu.sync_copy(zero_hbm, zbuf)

    fields = (fsrc, fdst, frel, fnorm)
    fields_hbm = (src_hbm, dst_hbm, rel_hbm, norm_hbm)

    def pass_body(p, _carry):
        ci = 2 * p + c
        lo = ci * CHUNK

        def zero_body(k, _):
            pltpu.sync_copy(
                zbuf,
                s_sh.at[pl.ds(
                    pl.multiple_of(s * ROWS_PER_TILE + k * ZROWS, ZROWS),
                    ZROWS)])
            return _

        lax.fori_loop(0, ROWS_PER_TILE // ZROWS, zero_body, None)

        @pl.when(s == NS - 1)
        def _():
            pltpu.sync_copy(zbuf.at[pl.ds(0, 8)],
                            s_sh.at[pl.ds(DUMMY_ROW, 8)])

        plsc.subcore_barrier()

        for fb, fh in zip(fields, fields_hbm):
            pltpu.async_copy(fh.at[pl.ds(ebase, SB)], fb.at[0], fsem.at[0])

        def fire(f):
            db = lax.rem(f, 2)

            @pl.when(f >= 2)
            def _():
                pltpu.make_async_copy(
                    pay.at[db],
                    s_sh.at[plsc.Indices(sidx.at[db],
                                         ignored_value=DUMMY_ROW)],
                    ssem.at[db]).wait()

            for g in range(EB // L):
                gidx[0, pl.ds(g * L, L)] = csrc[pl.ds(g * L, L)]
                sidx[db, pl.ds(g * L, L)] = csidx[pl.ds(g * L, L)]

            pltpu.async_copy(
                u_hbm.at[gidx.at[0]], rows, gsem).wait()

            for g in range(EB // L):
                nvv = cnorm[pl.ds(g * L, L)]
                for e2 in range(L):
                    e = g * L + e2
                    nv = jnp.full((L,), nvv[e2])
                    pay[db, e, pl.ds(0, L)] = rows[e, pl.ds(0, L)] * nv
                    pay[db, e, pl.ds(L, L)] = rows[e, pl.ds(L, L)] * nv

            pltpu.async_copy(
                pay.at[db],
                s_sh.at[plsc.Indices(sidx.at[db], ignored_value=DUMMY_ROW)],
                ssem.at[db], add=True)

        def vreg_body(i, _):
            sb = i // (SB // L)
            fdb = sb % 2

            @pl.when(lax.rem(i, SB // L) == 0)
            def _():
                for fb, fh in zip(fields, fields_hbm):
                    pltpu.make_async_copy(
                        fh.at[pl.ds(ebase + sb * SB, SB)], fb.at[fdb],
                        fsem.at[fdb]).wait()

                @pl.when(sb < N_SB - 1)
                def _():
                    nsb = sb + 1
                    for fb, fh in zip(fields, fields_hbm):
                        pltpu.async_copy(
                            fh.at[pl.ds(ebase + nsb * SB, SB)],
                            fb.at[nsb % 2], fsem.at[nsb % 2])

            off = lax.rem(i, SB // L) * L
            dstv = fdst[fdb, pl.ds(off, L)]
            relv = frel[fdb, pl.ds(off, L)]
            srcv = fsrc[fdb, pl.ds(off, L)]
            normv = fnorm[fdb, pl.ds(off, L)]
            inr = (dstv >= lo) & (dstv < lo + CHUNK)
            sx = (dstv - lo) * N_REL + relv

            pos = lax.rem(i, EB // L) * L
            csrc[pl.ds(pos, L)] = srcv
            csidx[pl.ds(pos, L)] = jnp.where(inr, sx, dummy16)
            cnorm[pl.ds(pos, L)] = normv

            @pl.when(lax.rem(i, EB // L) == EB // L - 1)
            def _():
                fire(i // (EB // L))

            return _

        lax.fori_loop(0, E_SLICE // L, vreg_body, None)

        for d in range(2):
            pltpu.make_async_copy(
                pay.at[d],
                s_sh.at[plsc.Indices(sidx.at[d], ignored_value=DUMMY_ROW)],
                ssem.at[d]).wait()

        plsc.subcore_barrier()

        pltpu.sync_copy(
            s_sh.at[pl.ds(s * ROWS_PER_TILE, ROWS_PER_TILE)],
            s_out.at[pl.ds(
                pl.multiple_of(ci * ROWS_PER_CHUNK + s * ROWS_PER_TILE, 8),
                ROWS_PER_TILE)])
        return _carry

    lax.fori_loop(0, N_PASSES, pass_body, None)


_sc_kernel = functools.partial(
    pl.kernel,
    out_type=jax.ShapeDtypeStruct((S_ROWS, ROW_W), jnp.float32),
    mesh=plsc.VectorSubcoreMesh(
        core_axis_name="c", subcore_axis_name="s", num_cores=NC,
        num_subcores=NS),
    scratch_types=[
        pltpu.VMEM((2, SB), jnp.int32),
        pltpu.VMEM((2, SB), jnp.int32),
        pltpu.VMEM((2, SB), jnp.int32),
        pltpu.VMEM((2, SB), jnp.float32),
        pltpu.VMEM((EB, ROW_W), jnp.float32),
        pltpu.VMEM((2, EB, ROW_W), jnp.float32),
        pltpu.VMEM((2, EB), jnp.int32),
        pltpu.VMEM((1, EB), jnp.int32),
        pltpu.VMEM((ZROWS, ROW_W), jnp.float32),
        pltpu.VMEM((EB + L,), jnp.int32),
        pltpu.VMEM((EB + 9 * L,), jnp.int32),
        pltpu.VMEM((EB + L,), jnp.float32),
        pltpu.SMEM((8,), jnp.int32),
        pltpu.VMEM_SHARED((ROWS_PER_CHUNK + 8, ROW_W), jnp.float32),
        pltpu.SemaphoreType.DMA((2,)),
        pltpu.SemaphoreType.DMA((2,)),
        pltpu.SemaphoreType.DMA,
    ],
    compiler_params=pltpu.CompilerParams(
        needs_layout_passes=False, use_tc_tiling_on_sc=False),
)(_sc_body)


def kernel(x, edge_index, rel_type, norm, weight, w_comp):
    src = edge_index[0].astype(jnp.int32)
    dst = edge_index[1].astype(jnp.int32)
    rel = rel_type.astype(jnp.int32)
    nrm = norm[:, 0].astype(jnp.float32)

    pad = E_PAD - N_EDGES
    src = jnp.concatenate([src, jnp.zeros((pad,), jnp.int32)])
    dst = jnp.concatenate([dst, jnp.full((pad,), DST_SENTINEL, jnp.int32)])
    rel = jnp.concatenate([rel, jnp.zeros((pad,), jnp.int32)])
    nrm = jnp.concatenate([nrm, jnp.zeros((pad,), jnp.float32)])

    u = pl.pallas_call(
        _u_tc_kernel,
        grid=(10,),
        in_specs=[
            pl.BlockSpec((1000, D), lambda i: (i, 0)),
            pl.BlockSpec((N_REL, N_BASES), lambda i: (0, 0)),
        ],
        out_specs=pl.BlockSpec((1000, ROW_W), lambda i: (i, 0)),
        out_shape=jax.ShapeDtypeStruct((N_NODES, ROW_W), jnp.float32),
    )(x, w_comp)

    zeros = jnp.zeros((ZROWS, ROW_W), jnp.float32)

    s_flat = _sc_kernel(u, src, dst, rel, nrm, zeros)

    s2 = s_flat.reshape(N_CHUNKS * CHUNK, N_REL * ROW_W)[:N_NODES]
    w_mat = weight.reshape(N_BASES * D, D)

    h = pl.pallas_call(
        _h_tc_kernel,
        grid=(10,),
        in_specs=[
            pl.BlockSpec((1000, 1024), lambda i: (i, 0)),
            pl.BlockSpec((1024, D), lambda i: (0, 0)),
        ],
        out_specs=pl.BlockSpec((1000, D), lambda i: (i, 0)),
        out_shape=jax.ShapeDtypeStruct((N_NODES, D), jnp.float32),
    )(s2, w_mat)

    return h


# speedup vs baseline: 1.3910x; 1.3910x over previous
import functools

import jax
import jax.numpy as jnp
from jax import lax
from jax.experimental import pallas as pl
from jax.experimental.pallas import tpu as pltpu
from jax.experimental.pallas import tpu_sc as plsc

N_NODES = 10000
N_EDGES = 320000
D = 128
N_REL = 32
N_BASES = 8

NC = 2
NS = 16
L = 16

CHUNK = 1280
N_PASSES = 4
N_CHUNKS = 8
ROW_W = 32
ROWS_PER_CHUNK = CHUNK * N_REL
DUMMY_ROW = ROWS_PER_CHUNK
S_ROWS = N_CHUNKS * ROWS_PER_CHUNK

E_PAD = 327680
E_SLICE = E_PAD // NS
SB = 512
EB = 128
N_BLOCKS = E_SLICE // EB
BLK_PER_SB = SB // EB
N_SB = E_SLICE // SB
ROWS_PER_TILE = ROWS_PER_CHUNK // NS
ZROWS = 64

DST_SENTINEL = 1 << 20


def _u_tc_kernel(x_ref, wc_ref, u_ref):
    for q in range(4):
        u_ref[:, q * N_BASES:(q + 1) * N_BASES] = jnp.dot(
            x_ref[:, q * 32:(q + 1) * 32], wc_ref[...],
            preferred_element_type=jnp.float32)


def _h_tc_kernel(s_ref, w_ref, h_ref):
    h_ref[...] = jnp.dot(s_ref[...], w_ref[...],
                         preferred_element_type=jnp.float32)


def _sc_body(u_hbm, src_hbm, dst_hbm, rel_hbm, norm_hbm, zero_hbm, s_out,
             fsrc, fdst, frel, fnorm, rows, pay, sidx, gidx, zbuf,
             csrc, csidx, cnorm, cnt_ref, u_sp, s_sh,
             fsem, ssem, gsem):
    c = lax.axis_index("c")
    s = lax.axis_index("s")
    ebase = pl.multiple_of(s * E_SLICE, SB)

    dummy16 = jnp.full((L,), DUMMY_ROW, jnp.int32)

    pltpu.sync_copy(zero_hbm, zbuf)

    @pl.when(s == 0)
    def _():
        pltpu.sync_copy(u_hbm, u_sp)

    fields = (fsrc, fdst, frel, fnorm)
    fields_hbm = (src_hbm, dst_hbm, rel_hbm, norm_hbm)

    def pass_body(p, _carry):
        ci = 2 * p + c
        lo = ci * CHUNK

        def zero_body(k, _):
            pltpu.sync_copy(
                zbuf,
                s_sh.at[pl.ds(
                    pl.multiple_of(s * ROWS_PER_TILE + k * ZROWS, ZROWS),
                    ZROWS)])
            return _

        lax.fori_loop(0, ROWS_PER_TILE // ZROWS, zero_body, None)

        @pl.when(s == NS - 1)
        def _():
            pltpu.sync_copy(zbuf.at[pl.ds(0, 8)],
                            s_sh.at[pl.ds(DUMMY_ROW, 8)])

        plsc.subcore_barrier()

        for fb, fh in zip(fields, fields_hbm):
            pltpu.async_copy(fh.at[pl.ds(ebase, SB)], fb.at[0], fsem.at[0])

        def fire(f):
            db = lax.rem(f, 2)

            @pl.when(f >= 2)
            def _():
                pltpu.make_async_copy(
                    pay.at[db],
                    s_sh.at[plsc.Indices(sidx.at[db],
                                         ignored_value=DUMMY_ROW)],
                    ssem.at[db]).wait()

            for g in range(EB // L):
                gidx[0, pl.ds(g * L, L)] = csrc[pl.ds(g * L, L)]
                sidx[db, pl.ds(g * L, L)] = csidx[pl.ds(g * L, L)]

            pltpu.async_copy(
                u_sp.at[gidx.at[0]], rows, gsem).wait()

            mhi = jnp.full((L,), -65536, jnp.int32)
            for g in range(EB // L):
                nvv = cnorm[pl.ds(g * L, L)]
                for e2 in range(L):
                    e = g * L + e2
                    nv = jnp.full((L,), nvv[e2])
                    pkv = rows[e, pl.ds(0, L)]
                    lo_f = plsc.bitcast(pkv << 16, jnp.float32)
                    hi_f = plsc.bitcast(pkv & mhi, jnp.float32)
                    pay[db, e, pl.ds(0, L)] = lo_f * nv
                    pay[db, e, pl.ds(L, L)] = hi_f * nv

            pltpu.async_copy(
                pay.at[db],
                s_sh.at[plsc.Indices(sidx.at[db], ignored_value=DUMMY_ROW)],
                ssem.at[db], add=True)

        def vreg_body(i, _):
            sb = i // (SB // L)
            fdb = sb % 2

            @pl.when(lax.rem(i, SB // L) == 0)
            def _():
                for fb, fh in zip(fields, fields_hbm):
                    pltpu.make_async_copy(
                        fh.at[pl.ds(ebase + sb * SB, SB)], fb.at[fdb],
                        fsem.at[fdb]).wait()

                @pl.when(sb < N_SB - 1)
                def _():
                    nsb = sb + 1
                    for fb, fh in zip(fields, fields_hbm):
                        pltpu.async_copy(
                            fh.at[pl.ds(ebase + nsb * SB, SB)],
                            fb.at[nsb % 2], fsem.at[nsb % 2])

            off = lax.rem(i, SB // L) * L
            dstv = fdst[fdb, pl.ds(off, L)]
            relv = frel[fdb, pl.ds(off, L)]
            srcv = fsrc[fdb, pl.ds(off, L)]
            normv = fnorm[fdb, pl.ds(off, L)]
            inr = (dstv >= lo) & (dstv < lo + CHUNK)
            sx = (dstv - lo) * N_REL + relv

            pos = lax.rem(i, EB // L) * L
            csrc[pl.ds(pos, L)] = srcv
            csidx[pl.ds(pos, L)] = jnp.where(inr, sx, dummy16)
            cnorm[pl.ds(pos, L)] = normv

            @pl.when(lax.rem(i, EB // L) == EB // L - 1)
            def _():
                fire(i // (EB // L))

            return _

        lax.fori_loop(0, E_SLICE // L, vreg_body, None)

        for d in range(2):
            pltpu.make_async_copy(
                pay.at[d],
                s_sh.at[plsc.Indices(sidx.at[d], ignored_value=DUMMY_ROW)],
                ssem.at[d]).wait()

        plsc.subcore_barrier()

        pltpu.sync_copy(
            s_sh.at[pl.ds(s * ROWS_PER_TILE, ROWS_PER_TILE)],
            s_out.at[pl.ds(
                pl.multiple_of(ci * ROWS_PER_CHUNK + s * ROWS_PER_TILE, 8),
                ROWS_PER_TILE)])
        return _carry

    lax.fori_loop(0, N_PASSES, pass_body, None)


_sc_kernel = functools.partial(
    pl.kernel,
    out_type=jax.ShapeDtypeStruct((S_ROWS, ROW_W), jnp.float32),
    mesh=plsc.VectorSubcoreMesh(
        core_axis_name="c", subcore_axis_name="s", num_cores=NC,
        num_subcores=NS),
    scratch_types=[
        pltpu.VMEM((2, SB), jnp.int32),
        pltpu.VMEM((2, SB), jnp.int32),
        pltpu.VMEM((2, SB), jnp.int32),
        pltpu.VMEM((2, SB), jnp.float32),
        pltpu.VMEM((EB, ROW_W // 2), jnp.int32),
        pltpu.VMEM((2, EB, ROW_W), jnp.float32),
        pltpu.VMEM((2, EB), jnp.int32),
        pltpu.VMEM((1, EB), jnp.int32),
        pltpu.VMEM((ZROWS, ROW_W), jnp.float32),
        pltpu.VMEM((EB + L,), jnp.int32),
        pltpu.VMEM((EB + 9 * L,), jnp.int32),
        pltpu.VMEM((EB + L,), jnp.float32),
        pltpu.SMEM((8,), jnp.int32),
        pltpu.VMEM_SHARED((N_NODES, ROW_W // 2), jnp.int32),
        pltpu.VMEM_SHARED((ROWS_PER_CHUNK + 8, ROW_W), jnp.float32),
        pltpu.SemaphoreType.DMA((2,)),
        pltpu.SemaphoreType.DMA((2,)),
        pltpu.SemaphoreType.DMA,
    ],
    compiler_params=pltpu.CompilerParams(
        needs_layout_passes=False, use_tc_tiling_on_sc=False),
)(_sc_body)


def kernel(x, edge_index, rel_type, norm, weight, w_comp):
    src = edge_index[0].astype(jnp.int32)
    dst = edge_index[1].astype(jnp.int32)
    rel = rel_type.astype(jnp.int32)
    nrm = norm[:, 0].astype(jnp.float32)

    pad = E_PAD - N_EDGES
    src = jnp.concatenate([src, jnp.zeros((pad,), jnp.int32)])
    dst = jnp.concatenate([dst, jnp.full((pad,), DST_SENTINEL, jnp.int32)])
    rel = jnp.concatenate([rel, jnp.zeros((pad,), jnp.int32)])
    nrm = jnp.concatenate([nrm, jnp.zeros((pad,), jnp.float32)])

    u = pl.pallas_call(
        _u_tc_kernel,
        grid=(10,),
        in_specs=[
            pl.BlockSpec((1000, D), lambda i: (i, 0)),
            pl.BlockSpec((N_REL, N_BASES), lambda i: (0, 0)),
        ],
        out_specs=pl.BlockSpec((1000, ROW_W), lambda i: (i, 0)),
        out_shape=jax.ShapeDtypeStruct((N_NODES, ROW_W), jnp.float32),
    )(x, w_comp)

    zeros = jnp.zeros((ZROWS, ROW_W), jnp.float32)

    u_pk = jax.lax.bitcast_convert_type(
        u.astype(jnp.bfloat16).reshape(N_NODES, ROW_W // 2, 2), jnp.int32)

    s_flat = _sc_kernel(u_pk, src, dst, rel, nrm, zeros)

    s2 = s_flat.reshape(N_CHUNKS * CHUNK, N_REL * ROW_W)[:N_NODES]
    perm = jnp.concatenate([jnp.arange(0, ROW_W, 2), jnp.arange(1, ROW_W, 2)])
    w_mat = weight.reshape(N_BASES * D, D).reshape(N_REL, ROW_W, D)[
        :, perm, :].reshape(N_BASES * D, D)

    h = pl.pallas_call(
        _h_tc_kernel,
        grid=(10,),
        in_specs=[
            pl.BlockSpec((1000, 1024), lambda i: (i, 0)),
            pl.BlockSpec((1024, D), lambda i: (0, 0)),
        ],
        out_specs=pl.BlockSpec((1000, D), lambda i: (i, 0)),
        out_shape=jax.ShapeDtypeStruct((N_NODES, D), jnp.float32),
    )(s2, w_mat)

    return h
